# baseline (speedup 1.0000x reference)
import functools

import jax
import jax.numpy as jnp
import numpy as np
from jax import lax
from jax.experimental import pallas as pl
from jax.experimental.pallas import tpu as pltpu

_TEMP = 0.1
_K = 64
_NCHUNK = 72
_BLOCK = 4096
_CHUNK = 128
_NEG_INF = np.float32(-np.inf)
_IMAX = np.int32(2**31 - 1)


def _main_kernel(vp_ref, q2_ref, se_ref, so_ref, idx_ref, scal_ref,
                 ss_ref, cm_ref, cand_ref, cid_ref, ms_ref, *, n, grid):
    g = pl.program_id(0)
    b = _BLOCK
    c = (2 * b) // _CHUNK

    @pl.when(g == 0)
    def _init():
        ms_ref[0] = _NEG_INF
        ms_ref[1] = jnp.float32(0.0)

    w = vp_ref[...]
    w2 = w * w
    n2a = jnp.sum(w2[:, :64], axis=1, keepdims=True)
    n2b = jnp.sum(w2[:, 64:], axis=1, keepdims=True)
    da = jnp.sqrt(n2a) + 1e-12
    db = jnp.sqrt(n2b) + 1e-12
    den = jnp.concatenate(
        [jnp.broadcast_to(da, (b, 64)), jnp.broadcast_to(db, (b, 64))], axis=1)
    nv = w / den
    s2 = lax.dot_general(q2_ref[...], nv,
                         (((1,), (1,)), ((), ())),
                         preferred_element_type=jnp.float32)
    se_ref[...] = s2[0, :]
    so_ref[...] = s2[1, :]

    sims2 = s2.reshape(c, _CHUNK)
    rowi = lax.broadcasted_iota(jnp.int32, (c, _CHUNK), 0)
    lane = lax.broadcasted_iota(jnp.int32, (c, _CHUNK), 1)
    half = c // 2
    plane = rowi // half
    col = g * b + (rowi % half) * _CHUNK + lane
    gidx2 = 2 * col + plane
    valid = gidx2 < n
    simsm = jnp.where(valid, sims2, _NEG_INF)

    logits = jnp.where(valid, sims2 / _TEMP, _NEG_INF)
    bm = jnp.max(logits)
    bs = jnp.sum(jnp.exp(logits - bm))
    m_old = ms_ref[0]
    s_old = ms_ref[1]
    m_new = jnp.maximum(m_old, bm)
    ms_ref[0] = m_new
    ms_ref[1] = s_old * jnp.exp(m_old - m_new) + bs * jnp.exp(bm - m_new)

    ss_ref[pl.ds(g * c, c), :] = simsm
    cm_ref[g, :] = jnp.max(simsm, axis=1)

    @pl.when(g == grid - 1)
    def _final():
        cm = cm_ref[...]
        cri = lax.broadcasted_iota(jnp.int32, (grid, c), 0)
        cci = lax.broadcasted_iota(jnp.int32, (grid, c), 1)
        cid = cri * c + cci

        def chunk_step(t, cmc):
            mt = jnp.max(cmc)
            it = jnp.min(jnp.where(cmc == mt, cid, _IMAX))
            cand_ref[t, :] = ss_ref[it, :]
            cid_ref[t, :] = jnp.full((_CHUNK,), it, jnp.int32)
            return jnp.where(cid == it, _NEG_INF, cmc)

        lax.fori_loop(0, _NCHUNK, chunk_step, cm)

        cand = cand_ref[...]
        lane2 = lax.broadcasted_iota(jnp.int32, (_NCHUNK, _CHUNK), 1)
        cidv = cid_ref[...]
        blk = cidv // c
        rin = cidv % c
        pl2 = rin // half
        col2 = blk * b + (rin % half) * _CHUNK + lane2
        gidx = 2 * col2 + pl2

        m_fin = ms_ref[0]
        s_fin = ms_ref[1]

        def topk_step(t, carry):
            candc, lp = carry
            vt = jnp.max(candc)
            it = jnp.min(jnp.where(candc == vt, gidx, _IMAX))
            idx_ref[t] = it
            pt = jnp.exp(vt / _TEMP - m_fin) / s_fin
            lp = lp + jnp.log(pt + 1e-10)
            return jnp.where(gidx == it, _NEG_INF, candc), lp

        _, lp = lax.fori_loop(0, _K, topk_step, (cand, jnp.float32(0.0)))
        scal_ref[0] = m_fin
        scal_ref[1] = s_fin
        scal_ref[2] = lp
        scal_ref[3] = jnp.float32(0.0)


def _probs_kernel(se_ref, so_ref, scal_ref, pe_ref, po_ref):
    m = scal_ref[0]
    s = scal_ref[1]
    pe_ref[...] = jnp.exp(se_ref[...] / _TEMP - m) / s
    po_ref[...] = jnp.exp(so_ref[...] / _TEMP - m) / s


def kernel(query, skill_vectors, k):
    n, d = skill_vectors.shape
    b = _BLOCK
    nh = n // 2
    vp = skill_vectors.reshape(nh, 2 * d)
    grid = (nh + b - 1) // b
    c = (2 * b) // _CHUNK

    q2 = jnp.zeros((2, 2 * d), jnp.float32)
    q2 = q2.at[0, :d].set(query).at[1, d:].set(query)

    se, so, idx, scal = pl.pallas_call(
        functools.partial(_main_kernel, n=n, grid=grid),
        grid=(grid,),
        in_specs=[
            pl.BlockSpec((b, 2 * d), lambda g: (g, 0)),
            pl.BlockSpec((2, 2 * d), lambda g: (0, 0)),
        ],
        out_specs=[
            pl.BlockSpec((b,), lambda g: (g,)),
            pl.BlockSpec((b,), lambda g: (g,)),
            pl.BlockSpec(memory_space=pltpu.SMEM),
            pl.BlockSpec(memory_space=pltpu.SMEM),
        ],
        out_shape=[
            jax.ShapeDtypeStruct((nh,), jnp.float32),
            jax.ShapeDtypeStruct((nh,), jnp.float32),
            jax.ShapeDtypeStruct((_K,), jnp.int32),
            jax.ShapeDtypeStruct((4,), jnp.float32),
        ],
        scratch_shapes=[
            pltpu.VMEM((grid * c, _CHUNK), jnp.float32),
            pltpu.VMEM((grid, c), jnp.float32),
            pltpu.VMEM((_NCHUNK, _CHUNK), jnp.float32),
            pltpu.VMEM((_NCHUNK, _CHUNK), jnp.int32),
            pltpu.SMEM((2,), jnp.float32),
        ],
    )(vp, q2)

    pe, po = pl.pallas_call(
        _probs_kernel,
        grid=(grid,),
        in_specs=[
            pl.BlockSpec((b,), lambda g: (g,)),
            pl.BlockSpec((b,), lambda g: (g,)),
            pl.BlockSpec(memory_space=pltpu.SMEM),
        ],
        out_specs=[
            pl.BlockSpec((b,), lambda g: (g,)),
            pl.BlockSpec((b,), lambda g: (g,)),
        ],
        out_shape=[
            jax.ShapeDtypeStruct((nh,), jnp.float32),
            jax.ShapeDtypeStruct((nh,), jnp.float32),
        ],
    )(se, so, scal)

    sims = jnp.stack([se, so], axis=1).reshape(n)
    probs = jnp.stack([pe, po], axis=1).reshape(n)
    log_probs = scal[2]
    indices = idx + (jnp.asarray(k, jnp.int32) - jnp.int32(_K))
    return (log_probs, probs, sims, indices)


# speedup vs baseline: 2.2380x; 2.2380x over previous
import functools

import jax
import jax.numpy as jnp
import numpy as np
from jax import lax
from jax.experimental import pallas as pl
from jax.experimental.pallas import tpu as pltpu

_TEMP = 0.1
_K = 64
_BLOCK = 16384
_CHUNK = 128
_NEG_INF = np.float32(-np.inf)
_IMAX = np.int32(2**31 - 1)


def _main_kernel(sv_ref, q_ref, sims_ref, idx_ref, scal_ref,
                 ss_ref, cm_ref, cand_ref, cid_ref, ms_ref, *, n, grid):
    g = pl.program_id(0)
    b = _BLOCK
    c = b // _CHUNK

    @pl.when(g == 0)
    def _init():
        ms_ref[0] = _NEG_INF
        ms_ref[1] = jnp.float32(0.0)

    v = sv_ref[...]
    n2 = jnp.sum(v * v, axis=1, keepdims=True)
    nv = v / (jnp.sqrt(n2) + 1e-12)
    s1 = lax.dot_general(q_ref[...], nv,
                         (((1,), (1,)), ((), ())),
                         preferred_element_type=jnp.float32)
    sims_ref[...] = s1

    sims2 = s1.reshape(c, _CHUNK)
    rowi = lax.broadcasted_iota(jnp.int32, (c, _CHUNK), 0)
    lane = lax.broadcasted_iota(jnp.int32, (c, _CHUNK), 1)
    gidx2 = g * b + rowi * _CHUNK + lane
    valid = gidx2 < n
    simsm = jnp.where(valid, sims2, _NEG_INF)

    logits = jnp.where(valid, sims2 / _TEMP, _NEG_INF)
    bm = jnp.max(logits)
    bs = jnp.sum(jnp.exp(logits - bm))
    m_old = ms_ref[0]
    s_old = ms_ref[1]
    m_new = jnp.maximum(m_old, bm)
    ms_ref[0] = m_new
    ms_ref[1] = s_old * jnp.exp(m_old - m_new) + bs * jnp.exp(bm - m_new)

    ss_ref[pl.ds(g * c, c), :] = simsm
    cm_ref[g, :] = jnp.max(simsm, axis=1)

    @pl.when(g == grid - 1)
    def _final():
        cm = cm_ref[...]
        cri = lax.broadcasted_iota(jnp.int32, (grid, c), 0)
        cci = lax.broadcasted_iota(jnp.int32, (grid, c), 1)
        cid = cri * c + cci

        def chunk_step(t, cmc):
            mt = jnp.max(cmc)
            it = jnp.min(jnp.where(cmc == mt, cid, _IMAX))
            cand_ref[t, :] = ss_ref[it, :]
            cid_ref[t, :] = jnp.full((_CHUNK,), it, jnp.int32)
            return jnp.where(cid == it, _NEG_INF, cmc)

        lax.fori_loop(0, _K, chunk_step, cm)

        cand = cand_ref[...]
        lane2 = lax.broadcasted_iota(jnp.int32, (_K, _CHUNK), 1)
        gidx = cid_ref[...] * _CHUNK + lane2

        m_fin = ms_ref[0]
        s_fin = ms_ref[1]

        def topk_step(t, carry):
            candc, lp = carry
            vt = jnp.max(candc)
            it = jnp.min(jnp.where(candc == vt, gidx, _IMAX))
            idx_ref[t] = it
            pt = jnp.exp(vt / _TEMP - m_fin) / s_fin
            lp = lp + jnp.log(pt + 1e-10)
            return jnp.where(gidx == it, _NEG_INF, candc), lp

        _, lp = lax.fori_loop(0, _K, topk_step, (cand, jnp.float32(0.0)))
        scal_ref[0] = m_fin
        scal_ref[1] = s_fin
        scal_ref[2] = lp
        scal_ref[3] = jnp.float32(0.0)


def _probs_kernel(sims_ref, scal_ref, probs_ref):
    probs_ref[...] = jnp.exp(sims_ref[...] / _TEMP - scal_ref[0]) / scal_ref[1]


def kernel(query, skill_vectors, k):
    n, d = skill_vectors.shape
    b = _BLOCK
    grid = (n + b - 1) // b
    c = b // _CHUNK

    sims2d, idx, scal = pl.pallas_call(
        functools.partial(_main_kernel, n=n, grid=grid),
        grid=(grid,),
        in_specs=[
            pl.BlockSpec((b, d), lambda g: (g, 0)),
            pl.BlockSpec((1, d), lambda g: (0, 0)),
        ],
        out_specs=[
            pl.BlockSpec((1, b), lambda g: (0, g)),
            pl.BlockSpec(memory_space=pltpu.SMEM),
            pl.BlockSpec(memory_space=pltpu.SMEM),
        ],
        out_shape=[
            jax.ShapeDtypeStruct((1, n), jnp.float32),
            jax.ShapeDtypeStruct((_K,), jnp.int32),
            jax.ShapeDtypeStruct((4,), jnp.float32),
        ],
        scratch_shapes=[
            pltpu.VMEM((grid * c, _CHUNK), jnp.float32),
            pltpu.VMEM((grid, c), jnp.float32),
            pltpu.VMEM((_K, _CHUNK), jnp.float32),
            pltpu.VMEM((_K, _CHUNK), jnp.int32),
            pltpu.SMEM((2,), jnp.float32),
        ],
    )(skill_vectors, query.reshape(1, d))

    probs2d = pl.pallas_call(
        _probs_kernel,
        grid=(grid,),
        in_specs=[
            pl.BlockSpec((1, b), lambda g: (0, g)),
            pl.BlockSpec(memory_space=pltpu.SMEM),
        ],
        out_specs=pl.BlockSpec((1, b), lambda g: (0, g)),
        out_shape=jax.ShapeDtypeStruct((1, n), jnp.float32),
    )(sims2d, scal)

    log_probs = scal[2]
    indices = idx + (jnp.asarray(k, jnp.int32) - jnp.int32(_K))
    return (log_probs, probs2d.reshape(n), sims2d.reshape(n), indices)


# speedup vs baseline: 2.3138x; 1.0339x over previous
import functools

import jax
import jax.numpy as jnp
import numpy as np
from jax import lax
from jax.experimental import pallas as pl
from jax.experimental.pallas import tpu as pltpu

_TEMP = 0.1
_K = 64
_BLOCK = 32768
_CHUNK = 128
_NEG_INF = np.float32(-np.inf)
_IMAX = np.int32(2**31 - 1)


def _main_kernel(sv_ref, q_ref, sims_ref, idx_ref, scal_ref,
                 ss_ref, cm_ref, cand_ref, cid_ref, ms_ref, *, n, grid):
    g = pl.program_id(0)
    b = _BLOCK
    c = b // _CHUNK

    @pl.when(g == 0)
    def _init():
        ms_ref[0] = _NEG_INF
        ms_ref[1] = jnp.float32(0.0)

    v = sv_ref[...]
    n2 = jnp.sum(v * v, axis=1, keepdims=True)
    nv = v / (jnp.sqrt(n2) + 1e-12)
    s1 = lax.dot_general(q_ref[...], nv,
                         (((1,), (1,)), ((), ())),
                         preferred_element_type=jnp.float32)
    sims_ref[...] = s1

    sims2 = s1.reshape(c, _CHUNK)
    rowi = lax.broadcasted_iota(jnp.int32, (c, _CHUNK), 0)
    lane = lax.broadcasted_iota(jnp.int32, (c, _CHUNK), 1)
    gidx2 = g * b + rowi * _CHUNK + lane
    valid = gidx2 < n
    simsm = jnp.where(valid, sims2, _NEG_INF)

    logits = jnp.where(valid, sims2 / _TEMP, _NEG_INF)
    bm = jnp.max(logits)
    bs = jnp.sum(jnp.exp(logits - bm))
    m_old = ms_ref[0]
    s_old = ms_ref[1]
    m_new = jnp.maximum(m_old, bm)
    ms_ref[0] = m_new
    ms_ref[1] = s_old * jnp.exp(m_old - m_new) + bs * jnp.exp(bm - m_new)

    ss_ref[pl.ds(g * c, c), :] = simsm
    cm_ref[g, :] = jnp.max(simsm, axis=1)

    @pl.when(g == grid - 1)
    def _final():
        cm = cm_ref[...]
        cri = lax.broadcasted_iota(jnp.int32, (grid, c), 0)
        cci = lax.broadcasted_iota(jnp.int32, (grid, c), 1)
        cid = cri * c + cci

        def chunk_step(t, cmc):
            mt = jnp.max(cmc)
            it = jnp.min(jnp.where(cmc == mt, cid, _IMAX))
            cand_ref[t, :] = ss_ref[it, :]
            cid_ref[t, :] = jnp.full((_CHUNK,), it, jnp.int32)
            return jnp.where(cid == it, _NEG_INF, cmc)

        lax.fori_loop(0, _K, chunk_step, cm)

        cand = cand_ref[...]
        lane2 = lax.broadcasted_iota(jnp.int32, (_K, _CHUNK), 1)
        gidx = cid_ref[...] * _CHUNK + lane2

        m_fin = ms_ref[0]
        s_fin = ms_ref[1]

        def topk_step(t, carry):
            candc, lp = carry
            vt = jnp.max(candc)
            it = jnp.min(jnp.where(candc == vt, gidx, _IMAX))
            idx_ref[t] = it
            pt = jnp.exp(vt / _TEMP - m_fin) / s_fin
            lp = lp + jnp.log(pt + 1e-10)
            return jnp.where(gidx == it, _NEG_INF, candc), lp

        _, lp = lax.fori_loop(0, _K, topk_step, (cand, jnp.float32(0.0)))
        scal_ref[0] = m_fin
        scal_ref[1] = s_fin
        scal_ref[2] = lp
        scal_ref[3] = jnp.float32(0.0)


def _probs_kernel(sims_ref, scal_ref, probs_ref):
    probs_ref[...] = jnp.exp(sims_ref[...] / _TEMP - scal_ref[0]) / scal_ref[1]


def kernel(query, skill_vectors, k):
    n, d = skill_vectors.shape
    b = _BLOCK
    grid = (n + b - 1) // b
    c = b // _CHUNK

    sims2d, idx, scal = pl.pallas_call(
        functools.partial(_main_kernel, n=n, grid=grid),
        grid=(grid,),
        in_specs=[
            pl.BlockSpec((b, d), lambda g: (g, 0)),
            pl.BlockSpec((1, d), lambda g: (0, 0)),
        ],
        out_specs=[
            pl.BlockSpec((1, b), lambda g: (0, g)),
            pl.BlockSpec(memory_space=pltpu.SMEM),
            pl.BlockSpec(memory_space=pltpu.SMEM),
        ],
        out_shape=[
            jax.ShapeDtypeStruct((1, n), jnp.float32),
            jax.ShapeDtypeStruct((_K,), jnp.int32),
            jax.ShapeDtypeStruct((4,), jnp.float32),
        ],
        scratch_shapes=[
            pltpu.VMEM((grid * c, _CHUNK), jnp.float32),
            pltpu.VMEM((grid, c), jnp.float32),
            pltpu.VMEM((_K, _CHUNK), jnp.float32),
            pltpu.VMEM((_K, _CHUNK), jnp.int32),
            pltpu.SMEM((2,), jnp.float32),
        ],
    )(skill_vectors, query.reshape(1, d))

    probs2d = pl.pallas_call(
        _probs_kernel,
        grid=(grid,),
        in_specs=[
            pl.BlockSpec((1, b), lambda g: (0, g)),
            pl.BlockSpec(memory_space=pltpu.SMEM),
        ],
        out_specs=pl.BlockSpec((1, b), lambda g: (0, g)),
        out_shape=jax.ShapeDtypeStruct((1, n), jnp.float32),
    )(sims2d, scal)

    log_probs = scal[2]
    indices = idx + (jnp.asarray(k, jnp.int32) - jnp.int32(_K))
    return (log_probs, probs2d.reshape(n), sims2d.reshape(n), indices)
